# baseline (speedup 1.0000x reference)
import functools

import jax
import jax.numpy as jnp
from jax import lax
from jax.experimental import pallas as pl
from jax.experimental.pallas import tpu as pltpu
from jax.experimental.pallas import tpu_sc as plsc

_N = 10000
_E = 320000
_D = 128
_NC = 2
_NS = 16
_NW = _NC * _NS
_EPW = _E // _NW
_C = 80
_NCHUNK = _EPW // _C
_NB = 3
_RPT = _N // _NS
_WBR = 624
_TC_BR = 1000
_RTC = 6416
_RSC = _N - _RTC
_NBLK = _RSC // 8
_BPW = _NBLK // _NW
_HW = 4992


def _tanh_body(x_ref, o_ref):
    o_ref[...] = jnp.tanh(x_ref[...])


def _add_body(a_ref, b_ref, o_ref):
    o_ref[...] = a_ref[0] + b_ref[0]


_sc_mesh = plsc.VectorSubcoreMesh(core_axis_name="c", subcore_axis_name="s")


@functools.partial(
    pl.kernel,
    out_type=jax.ShapeDtypeStruct((_NBLK, 8, 2 * _HW), jnp.float32),
    mesh=_sc_mesh,
    compiler_params=pltpu.CompilerParams(needs_layout_passes=False),
    scratch_types=[
        [pltpu.VMEM((8, _HW), jnp.float32)] * 2,
        [pltpu.SemaphoreType.DMA] * 2,
        [pltpu.SemaphoreType.DMA] * 2,
    ],
)
def _sc_pad(w_hbm, out_hbm, bufs, rsem, wsem):
    cc = lax.axis_index("c")
    ss = lax.axis_index("s")
    base = (cc * _NS + ss) * _BPW

    def _rd_args(st, b):
        blk = base + (st >> 1)
        h = st & 1
        return (w_hbm.at[pl.ds(_RTC + blk * 8, 8), pl.ds(h * _HW, _HW)],
                bufs[b], rsem[b])

    def _wr_args(st, b):
        blk = base + (st >> 1)
        h = st & 1
        return (bufs[b], out_hbm.at[blk, :, pl.ds(h * _HW, _HW)], wsem[b])

    pltpu.async_copy(*_rd_args(0, 0))

    def _step(i, carry):
        for u in range(2):
            st = 2 * i + u
            b = u
            ob = 1 - u

            @pl.when(st < 2 * _BPW - 1)
            def _pf():
                @pl.when(st >= 1)
                def _ww():
                    pltpu.make_async_copy(*_wr_args(st - 1, ob)).wait()

                pltpu.async_copy(*_rd_args(st + 1, ob))

            pltpu.make_async_copy(*_rd_args(st, b)).wait()
            pltpu.async_copy(*_wr_args(st, b))
        return carry

    lax.fori_loop(0, _BPW, _step, 0)
    pltpu.make_async_copy(*_wr_args(2 * _BPW - 2, 0)).wait()
    pltpu.make_async_copy(*_wr_args(2 * _BPW - 1, 1)).wait()


@functools.partial(
    pl.kernel,
    out_type=jax.ShapeDtypeStruct((_NC, _N, _D), jnp.float32),
    mesh=_sc_mesh,
    compiler_params=pltpu.CompilerParams(needs_layout_passes=False),
    scratch_types=[
        pltpu.VMEM((_EPW,), jnp.int32),
        [pltpu.VMEM((_C,), jnp.int32)] * _NB,
        [pltpu.VMEM((_C,), jnp.int32)] * _NB,
        [pltpu.VMEM((_C,), jnp.int32)] * _NB,
        [pltpu.VMEM((_C,), jnp.int32)] * _NB,
        [pltpu.VMEM((_C,), jnp.int32)] * _NB,
        [pltpu.VMEM((_C,), jnp.int32)] * _NB,
        [pltpu.VMEM((_C,), jnp.float32)] * _NB,
        [pltpu.VMEM((_C,), jnp.float32)] * _NB,
        [pltpu.VMEM((_C,), jnp.float32)] * _NB,
        [pltpu.VMEM((_C,), jnp.float32)] * _NB,
        [pltpu.VMEM((_C, _D), jnp.float32)] * _NB,
        pltpu.VMEM_SHARED((_N, _D), jnp.float32),
        [pltpu.SemaphoreType.DMA] * _NB,
        [pltpu.SemaphoreType.DMA] * _NB,
        [pltpu.SemaphoreType.DMA] * _NB,
    ],
)
def _sc_scatter(t_hbm, widx_hbm, wtc_hbm, wsc_hbm, wtl_hbm, out_hbm,
                widx_v, srcs, dsts, ptc, psc, ptl, sel, gtc, gsc, gtl, wvs,
                rowss, acc_sh, gw, gt, sc):
    c = lax.axis_index("c")
    s = lax.axis_index("s")
    wid = c * _NS + s

    pltpu.sync_copy(widx_hbm.at[pl.ds(wid * _EPW, _EPW)], widx_v)

    def _zrow(e, carry):
        z = jnp.zeros((16,), jnp.float32)
        for j in range(_D // 16):
            rowss[0][e, pl.ds(j * 16, 16)] = z
        return carry

    lax.fori_loop(0, _C, _zrow, 0)
    zbase = s * _RPT
    for r in range(_RPT // _C):
        pltpu.sync_copy(rowss[0], acc_sh.at[pl.ds(zbase + r * _C, _C)])
    _rem = _RPT - (_RPT // _C) * _C
    pltpu.sync_copy(rowss[0].at[pl.ds(0, _rem)],
                    acc_sh.at[pl.ds(zbase + (_RPT // _C) * _C, _rem)])
    plsc.subcore_barrier()

    def _issue(k, b):
        for i in range(_C // 16):
            wv16 = widx_v[pl.ds(k * _C + i * 16, 16)]
            s16 = wv16 // _N
            d16 = wv16 - s16 * _N
            sl = pl.ds(i * 16, 16)
            srcs[b][sl] = s16
            dsts[b][sl] = d16
            stc = jnp.minimum(s16, _RTC - 1)
            ptc[b][sl] = ((stc >> 3) * (79 * 1024) + ((d16 >> 7) << 10)
                          + ((stc & 7) << 7) + (d16 & 127))
            ssc = jnp.maximum(s16 - _RTC, 0)
            dsc = jnp.minimum(d16, 2 * _HW - 1)
            psc[b][sl] = ((ssc >> 3) * (78 * 1024) + ((dsc >> 7) << 10)
                          + ((ssc & 7) << 7) + (dsc & 127))
            ptl[b][sl] = (ssc << 4) + jnp.maximum(d16 - 2 * _HW, 0)
            sel[b][sl] = jnp.where(
                s16 < _RTC, 0, jnp.where(d16 < 2 * _HW, 1, 2))
        pltpu.async_copy(wtc_hbm.at[ptc[b]], gtc[b], gw[b])
        pltpu.async_copy(wsc_hbm.at[psc[b]], gsc[b], gw[b])
        pltpu.async_copy(wtl_hbm.at[ptl[b]], gtl[b], gw[b])
        pltpu.async_copy(t_hbm.at[srcs[b]], rowss[b], gt[b])

    def _wait_gathers(k, b):
        pltpu.make_async_copy(wtc_hbm.at[ptc[b]], gtc[b], gw[b]).wait()
        pltpu.make_async_copy(wsc_hbm.at[psc[b]], gsc[b], gw[b]).wait()
        pltpu.make_async_copy(wtl_hbm.at[ptl[b]], gtl[b], gw[b]).wait()
        pltpu.make_async_copy(t_hbm.at[srcs[b]], rowss[b], gt[b]).wait()
        for i in range(_C // 16):
            sl = pl.ds(i * 16, 16)
            se = sel[b][sl]
            wvs[b][sl] = jnp.where(
                se == 0, gtc[b][sl],
                jnp.where(se == 1, gsc[b][sl], gtl[b][sl]))

    def _scale(b):
        def _srow(e4, cc):
            for v in range(4):
                e = e4 * 4 + v
                eidx = jnp.full((16,), e, jnp.int32)
                wsc = plsc.load_gather(wvs[b], [eidx])
                for j in range(_D // 16):
                    sl = pl.ds(j * 16, 16)
                    rowss[b][e, sl] = rowss[b][e, sl] * wsc
            return cc

        lax.fori_loop(0, _C // 4, _srow, 0)

    def _scatter(k, b):
        pltpu.async_copy(rowss[b], acc_sh.at[dsts[b]], sc[b], add=True)

    def _wait_scatter(k, b):
        pltpu.make_async_copy(rowss[b], acc_sh.at[dsts[b]], sc[b]).wait()

    _issue(0, 0)

    def _group(i, carry):
        for u in range(_NB):
            k = i * _NB + u
            b = u
            b1 = (u + 1) % _NB

            @pl.when(k <= _NCHUNK - 1)
            def _body():
                @pl.when(jnp.logical_and(k >= 2, k <= _NCHUNK - 2))
                def _free():
                    _wait_scatter(k - 2, b1)

                @pl.when(k <= _NCHUNK - 2)
                def _prefetch():
                    _issue(k + 1, b1)

                _wait_gathers(k, b)
                _scale(b)
                _scatter(k, b)

        return carry

    lax.fori_loop(0, (_NCHUNK + _NB - 1) // _NB, _group, 0)
    for kk in range(_NCHUNK - 3, _NCHUNK):
        _wait_scatter(kk, kk % _NB)

    plsc.subcore_barrier()

    wb = s * _WBR
    pltpu.sync_copy(acc_sh.at[pl.ds(wb, _WBR)],
                    out_hbm.at[c, pl.ds(wb, _WBR)])

    @pl.when(s == _NS - 1)
    def _tail():
        pltpu.sync_copy(acc_sh.at[pl.ds(_NS * _WBR, _N - _NS * _WBR)],
                        out_hbm.at[c, pl.ds(_NS * _WBR, _N - _NS * _WBR)])


def kernel(values, edge_index, w):
    widx = edge_index[0] * _N + edge_index[1]
    wpad = jnp.pad(w[:_RTC], ((0, 0), (0, 112)))
    wflat_tc = (wpad.reshape(_RTC // 8, 8, 79, 128)
                .transpose(0, 2, 1, 3)
                .reshape(_RTC // 8 * 79 * 1024))
    wsc3 = _sc_pad(w)
    wflat_sc = (wsc3.reshape(_NBLK, 8, 78, 128)
                .transpose(0, 2, 1, 3)
                .reshape(_NBLK * 78 * 1024))
    wtail = w[_RTC:, 2 * _HW:].reshape(_RSC * 16)

    t = pl.pallas_call(
        _tanh_body,
        grid=(_N // _TC_BR,),
        in_specs=[pl.BlockSpec((_TC_BR, _D), lambda i: (i, 0))],
        out_specs=pl.BlockSpec((_TC_BR, _D), lambda i: (i, 0)),
        out_shape=jax.ShapeDtypeStruct((_N, _D), jnp.float32),
    )(values)

    partials = _sc_scatter(t, widx, wflat_tc, wflat_sc, wtail)

    out = pl.pallas_call(
        _add_body,
        grid=(_N // _TC_BR,),
        in_specs=[
            pl.BlockSpec((1, _TC_BR, _D), lambda i: (0, i, 0)),
            pl.BlockSpec((1, _TC_BR, _D), lambda i: (1, i, 0)),
        ],
        out_specs=pl.BlockSpec((_TC_BR, _D), lambda i: (i, 0)),
        out_shape=jax.ShapeDtypeStruct((_N, _D), jnp.float32),
    )(partials, partials)
    return out


# speedup vs baseline: 2.3026x; 2.3026x over previous
import functools

import jax
import jax.numpy as jnp
from jax import lax
from jax.experimental import pallas as pl
from jax.experimental.pallas import tpu as pltpu
from jax.experimental.pallas import tpu_sc as plsc

_N = 10000
_E = 320000
_D = 128
_NC = 2
_NS = 16
_NW = _NC * _NS
_EPW = _E // _NW
_C = 80
_NCHUNK = _EPW // _C
_NB = 3
_RPT = _N // _NS
_WBR = 624
_TC_BR = 1000
_RTC = 6416
_RSC = _N - _RTC
_NBLK = _RSC // 8
_BPW = _NBLK // _NW
_HW = 4992


def _tanh_body(x_ref, o_ref):
    o_ref[...] = jnp.tanh(x_ref[...])


def _add_body(a_ref, b_ref, o_ref):
    o_ref[...] = a_ref[0] + b_ref[0]


_sc_mesh = plsc.VectorSubcoreMesh(core_axis_name="c", subcore_axis_name="s")


@functools.partial(
    pl.kernel,
    out_type=jax.ShapeDtypeStruct((_NBLK, 8, 2 * _HW), jnp.float32),
    mesh=_sc_mesh,
    compiler_params=pltpu.CompilerParams(needs_layout_passes=False),
    scratch_types=[
        [pltpu.VMEM((8, _HW), jnp.float32)] * 2,
        [pltpu.SemaphoreType.DMA] * 2,
        [pltpu.SemaphoreType.DMA] * 2,
    ],
)
def _sc_pad(w_hbm, out_hbm, bufs, rsem, wsem):
    cc = lax.axis_index("c")
    ss = lax.axis_index("s")
    base = (cc * _NS + ss) * _BPW

    def _rd_args(st, b):
        blk = base + (st >> 1)
        h = st & 1
        return (w_hbm.at[pl.ds(_RTC + blk * 8, 8), pl.ds(h * _HW, _HW)],
                bufs[b], rsem[b])

    def _wr_args(st, b):
        blk = base + (st >> 1)
        h = st & 1
        return (bufs[b], out_hbm.at[blk, :, pl.ds(h * _HW, _HW)], wsem[b])

    pltpu.async_copy(*_rd_args(0, 0))

    def _step(i, carry):
        for u in range(2):
            st = 2 * i + u
            b = u
            ob = 1 - u

            @pl.when(st < 2 * _BPW - 1)
            def _pf():
                @pl.when(st >= 1)
                def _ww():
                    pltpu.make_async_copy(*_wr_args(st - 1, ob)).wait()

                pltpu.async_copy(*_rd_args(st + 1, ob))

            pltpu.make_async_copy(*_rd_args(st, b)).wait()
            pltpu.async_copy(*_wr_args(st, b))
        return carry

    lax.fori_loop(0, _BPW, _step, 0)
    pltpu.make_async_copy(*_wr_args(2 * _BPW - 2, 0)).wait()
    pltpu.make_async_copy(*_wr_args(2 * _BPW - 1, 1)).wait()


@functools.partial(
    pl.kernel,
    out_type=jax.ShapeDtypeStruct((_NC, _N, _D), jnp.float32),
    mesh=_sc_mesh,
    compiler_params=pltpu.CompilerParams(needs_layout_passes=False),
    scratch_types=[
        pltpu.VMEM((_EPW,), jnp.int32),
        [pltpu.VMEM((_C,), jnp.int32)] * _NB,
        [pltpu.VMEM((_C,), jnp.int32)] * _NB,
        [pltpu.VMEM((_C,), jnp.int32)] * _NB,
        [pltpu.VMEM((_C,), jnp.int32)] * _NB,
        [pltpu.VMEM((_C,), jnp.int32)] * _NB,
        [pltpu.VMEM((_C,), jnp.int32)] * _NB,
        [pltpu.VMEM((_C,), jnp.float32)] * _NB,
        [pltpu.VMEM((_C,), jnp.float32)] * _NB,
        [pltpu.VMEM((_C,), jnp.float32)] * _NB,
        [pltpu.VMEM((_C,), jnp.float32)] * _NB,
        [pltpu.VMEM((_C, _D), jnp.float32)] * _NB,
        pltpu.VMEM_SHARED((_N, _D), jnp.float32),
        [pltpu.SemaphoreType.DMA] * _NB,
        [pltpu.SemaphoreType.DMA] * _NB,
        [pltpu.SemaphoreType.DMA] * _NB,
    ],
)
def _sc_scatter(t_hbm, widx_hbm, wtc_hbm, wsc_hbm, wtl_hbm, out_hbm,
                widx_v, srcs, dsts, ptc, psc, ptl, sel, gtc, gsc, gtl, wvs,
                rowss, acc_sh, gw, gt, sc):
    c = lax.axis_index("c")
    s = lax.axis_index("s")
    wid = c * _NS + s

    pltpu.sync_copy(widx_hbm.at[pl.ds(wid * _EPW, _EPW)], widx_v)

    def _zrow(e, carry):
        z = jnp.zeros((16,), jnp.float32)
        for j in range(_D // 16):
            rowss[0][e, pl.ds(j * 16, 16)] = z
        return carry

    lax.fori_loop(0, _C, _zrow, 0)
    zbase = s * _RPT
    for r in range(_RPT // _C):
        pltpu.sync_copy(rowss[0], acc_sh.at[pl.ds(zbase + r * _C, _C)])
    _rem = _RPT - (_RPT // _C) * _C
    pltpu.sync_copy(rowss[0].at[pl.ds(0, _rem)],
                    acc_sh.at[pl.ds(zbase + (_RPT // _C) * _C, _rem)])
    plsc.subcore_barrier()

    def _issue(k, b):
        for i in range(_C // 16):
            wv16 = widx_v[pl.ds(k * _C + i * 16, 16)]
            s16 = wv16 // _N
            d16 = wv16 - s16 * _N
            sl = pl.ds(i * 16, 16)
            srcs[b][sl] = s16
            dsts[b][sl] = d16
            in_tc = s16 < _RTC
            in_tl = jnp.logical_and(jnp.logical_not(in_tc),
                                    d16 >= 2 * _HW)
            ssc = s16 - _RTC
            ptc[b][sl] = jnp.where(
                in_tc,
                (s16 >> 3) * (79 * 1024) + ((d16 >> 7) << 10)
                + ((s16 & 7) << 7) + (d16 & 127),
                wv16 & 0x1FFFFFF)
            psc[b][sl] = jnp.where(
                jnp.logical_or(in_tc, in_tl),
                wv16 & 0x1FFFFFF,
                (ssc >> 3) * (78 * 1024) + ((d16 >> 7) << 10)
                + ((ssc & 7) << 7) + (d16 & 127))
            ptl[b][sl] = jnp.where(in_tl, (ssc << 7) + (d16 - 2 * _HW),
                                   wv16 & 0xFFFF)
            sel[b][sl] = jnp.where(in_tc, 0, jnp.where(in_tl, 2, 1))
        pltpu.async_copy(wtc_hbm.at[ptc[b]], gtc[b], gw[b])
        pltpu.async_copy(wsc_hbm.at[psc[b]], gsc[b], gw[b])
        pltpu.async_copy(wtl_hbm.at[ptl[b]], gtl[b], gw[b])
        pltpu.async_copy(t_hbm.at[srcs[b]], rowss[b], gt[b])

    def _wait_gathers(k, b):
        pltpu.make_async_copy(wtc_hbm.at[ptc[b]], gtc[b], gw[b]).wait()
        pltpu.make_async_copy(wsc_hbm.at[psc[b]], gsc[b], gw[b]).wait()
        pltpu.make_async_copy(wtl_hbm.at[ptl[b]], gtl[b], gw[b]).wait()
        pltpu.make_async_copy(t_hbm.at[srcs[b]], rowss[b], gt[b]).wait()
        for i in range(_C // 16):
            sl = pl.ds(i * 16, 16)
            se = sel[b][sl]
            wvs[b][sl] = jnp.where(
                se == 0, gtc[b][sl],
                jnp.where(se == 1, gsc[b][sl], gtl[b][sl]))

    def _scale(b):
        def _srow(e4, cc):
            for v in range(4):
                e = e4 * 4 + v
                eidx = jnp.full((16,), e, jnp.int32)
                wsc = plsc.load_gather(wvs[b], [eidx])
                for j in range(_D // 16):
                    sl = pl.ds(j * 16, 16)
                    rowss[b][e, sl] = rowss[b][e, sl] * wsc
            return cc

        lax.fori_loop(0, _C // 4, _srow, 0)

    def _scatter(k, b):
        pltpu.async_copy(rowss[b], acc_sh.at[dsts[b]], sc[b], add=True)

    def _wait_scatter(k, b):
        pltpu.make_async_copy(rowss[b], acc_sh.at[dsts[b]], sc[b]).wait()

    _issue(0, 0)

    def _group(i, carry):
        for u in range(_NB):
            k = i * _NB + u
            b = u
            b1 = (u + 1) % _NB

            @pl.when(k <= _NCHUNK - 1)
            def _body():
                @pl.when(jnp.logical_and(k >= 2, k <= _NCHUNK - 2))
                def _free():
                    _wait_scatter(k - 2, b1)

                @pl.when(k <= _NCHUNK - 2)
                def _prefetch():
                    _issue(k + 1, b1)

                _wait_gathers(k, b)
                _scale(b)
                _scatter(k, b)

        return carry

    lax.fori_loop(0, (_NCHUNK + _NB - 1) // _NB, _group, 0)
    for kk in range(_NCHUNK - 3, _NCHUNK):
        _wait_scatter(kk, kk % _NB)

    plsc.subcore_barrier()

    wb = s * _WBR
    pltpu.sync_copy(acc_sh.at[pl.ds(wb, _WBR)],
                    out_hbm.at[c, pl.ds(wb, _WBR)])

    @pl.when(s == _NS - 1)
    def _tail():
        pltpu.sync_copy(acc_sh.at[pl.ds(_NS * _WBR, _N - _NS * _WBR)],
                        out_hbm.at[c, pl.ds(_NS * _WBR, _N - _NS * _WBR)])


def kernel(values, edge_index, w):
    widx = edge_index[0] * _N + edge_index[1]
    wpad = jnp.pad(w[:_RTC], ((0, 0), (0, 112)))
    wflat_tc = (wpad.reshape(_RTC // 8, 8, 79, 128)
                .transpose(0, 2, 1, 3)
                .reshape(_RTC // 8 * 79 * 1024))
    wsc3 = _sc_pad(w)
    wflat_sc = (wsc3.reshape(_NBLK, 8, 78, 128)
                .transpose(0, 2, 1, 3)
                .reshape(_NBLK * 78 * 1024))
    wtail = (jnp.pad(w[_RTC:, 2 * _HW:], ((0, 0), (0, 112)))
             .reshape(_RSC * 128))

    t = pl.pallas_call(
        _tanh_body,
        grid=(_N // _TC_BR,),
        in_specs=[pl.BlockSpec((_TC_BR, _D), lambda i: (i, 0))],
        out_specs=pl.BlockSpec((_TC_BR, _D), lambda i: (i, 0)),
        out_shape=jax.ShapeDtypeStruct((_N, _D), jnp.float32),
    )(values)

    partials = _sc_scatter(t, widx, wflat_tc, wflat_sc, wtail)

    out = pl.pallas_call(
        _add_body,
        grid=(_N // _TC_BR,),
        in_specs=[
            pl.BlockSpec((1, _TC_BR, _D), lambda i: (0, i, 0)),
            pl.BlockSpec((1, _TC_BR, _D), lambda i: (1, i, 0)),
        ],
        out_specs=pl.BlockSpec((_TC_BR, _D), lambda i: (i, 0)),
        out_shape=jax.ShapeDtypeStruct((_N, _D), jnp.float32),
    )(partials, partials)
    return out


# speedup vs baseline: 3.4233x; 1.4867x over previous
import functools

import jax
import jax.numpy as jnp
from jax import lax
from jax.experimental import pallas as pl
from jax.experimental.pallas import tpu as pltpu
from jax.experimental.pallas import tpu_sc as plsc

_N = 10000
_E = 320000
_D = 128
_NC = 2
_NS = 16
_NW = _NC * _NS
_EPW = _E // _NW
_C = 80
_NCHUNK = _EPW // _C
_NB = 3
_RPT = _N // _NS
_WBR = 624
_TC_BR = 1000


def _tanh_body(x_ref, o_ref):
    o_ref[...] = jnp.tanh(x_ref[...])


def _add_body(a_ref, b_ref, o_ref):
    o_ref[...] = a_ref[0] + b_ref[0]


_sc_mesh = plsc.VectorSubcoreMesh(core_axis_name="c", subcore_axis_name="s")


@functools.partial(
    pl.kernel,
    out_type=jax.ShapeDtypeStruct((_NC, _N, _D), jnp.float32),
    mesh=_sc_mesh,
    compiler_params=pltpu.CompilerParams(needs_layout_passes=False),
    scratch_types=[
        pltpu.VMEM((_EPW,), jnp.int32),
        [pltpu.VMEM((_C,), jnp.int32)] * _NB,
        [pltpu.VMEM((_C,), jnp.int32)] * _NB,
        [pltpu.VMEM((_C,), jnp.int32)] * _NB,
        [pltpu.VMEM((_C,), jnp.float32)] * _NB,
        [pltpu.VMEM((_C, _D), jnp.float32)] * _NB,
        pltpu.VMEM_SHARED((_N, _D), jnp.float32),
        [pltpu.SemaphoreType.DMA] * _NB,
        [pltpu.SemaphoreType.DMA] * _NB,
        [pltpu.SemaphoreType.DMA] * _NB,
    ],
)
def _sc_scatter(t_hbm, widx_hbm, wflat_hbm, out_hbm,
                widx_v, srcs, dsts, wps, wvs, rowss, acc_sh, gw, gt, sc):
    c = lax.axis_index("c")
    s = lax.axis_index("s")
    wid = c * _NS + s

    pltpu.sync_copy(widx_hbm.at[pl.ds(wid * _EPW, _EPW)], widx_v)

    def _zrow(e, carry):
        z = jnp.zeros((16,), jnp.float32)
        for j in range(_D // 16):
            rowss[0][e, pl.ds(j * 16, 16)] = z
        return carry

    lax.fori_loop(0, _C, _zrow, 0)
    zbase = s * _RPT
    for r in range(_RPT // _C):
        pltpu.sync_copy(rowss[0], acc_sh.at[pl.ds(zbase + r * _C, _C)])
    _rem = _RPT - (_RPT // _C) * _C
    pltpu.sync_copy(rowss[0].at[pl.ds(0, _rem)],
                    acc_sh.at[pl.ds(zbase + (_RPT // _C) * _C, _rem)])
    plsc.subcore_barrier()

    def _issue(k, b):
        for i in range(_C // 16):
            wv16 = widx_v[pl.ds(k * _C + i * 16, 16)]
            s16 = wv16 // _N
            d16 = wv16 - s16 * _N
            sl = pl.ds(i * 16, 16)
            srcs[b][sl] = s16
            dsts[b][sl] = d16
            wps[b][sl] = ((s16 >> 3) * (79 * 1024) + ((d16 >> 7) << 10)
                          + ((s16 & 7) << 7) + (d16 & 127))
        pltpu.async_copy(wflat_hbm.at[wps[b]], wvs[b], gw[b])
        pltpu.async_copy(t_hbm.at[srcs[b]], rowss[b], gt[b])

    def _wait_gathers(k, b):
        pltpu.make_async_copy(wflat_hbm.at[wps[b]], wvs[b], gw[b]).wait()
        pltpu.make_async_copy(t_hbm.at[srcs[b]], rowss[b], gt[b]).wait()

    def _scale(b):
        def _srow(e4, cc):
            for v in range(4):
                e = e4 * 4 + v
                eidx = jnp.full((16,), e, jnp.int32)
                wsc = plsc.load_gather(wvs[b], [eidx])
                for j in range(_D // 16):
                    sl = pl.ds(j * 16, 16)
                    rowss[b][e, sl] = rowss[b][e, sl] * wsc
            return cc

        lax.fori_loop(0, _C // 4, _srow, 0)

    def _scatter(k, b):
        pltpu.async_copy(rowss[b], acc_sh.at[dsts[b]], sc[b], add=True)

    def _wait_scatter(k, b):
        pltpu.make_async_copy(rowss[b], acc_sh.at[dsts[b]], sc[b]).wait()

    _issue(0, 0)

    def _group(i, carry):
        for u in range(_NB):
            k = i * _NB + u
            b = u
            b1 = (u + 1) % _NB

            @pl.when(k <= _NCHUNK - 1)
            def _body():
                @pl.when(jnp.logical_and(k >= 2, k <= _NCHUNK - 2))
                def _free():
                    _wait_scatter(k - 2, b1)

                @pl.when(k <= _NCHUNK - 2)
                def _prefetch():
                    _issue(k + 1, b1)

                _wait_gathers(k, b)
                _scale(b)
                _scatter(k, b)

        return carry

    lax.fori_loop(0, (_NCHUNK + _NB - 1) // _NB, _group, 0)
    for kk in range(_NCHUNK - 3, _NCHUNK):
        _wait_scatter(kk, kk % _NB)

    plsc.subcore_barrier()

    wb = s * _WBR
    pltpu.sync_copy(acc_sh.at[pl.ds(wb, _WBR)],
                    out_hbm.at[c, pl.ds(wb, _WBR)])

    @pl.when(s == _NS - 1)
    def _tail():
        pltpu.sync_copy(acc_sh.at[pl.ds(_NS * _WBR, _N - _NS * _WBR)],
                        out_hbm.at[c, pl.ds(_NS * _WBR, _N - _NS * _WBR)])


def kernel(values, edge_index, w):
    widx = edge_index[0] * _N + edge_index[1]
    wpad = jnp.pad(w, ((0, 0), (0, 112)))
    wflat = (wpad.reshape(_N // 8, 8, 79, 128)
             .transpose(0, 2, 1, 3)
             .reshape(_N // 8 * 79 * 8 * 128))

    t = pl.pallas_call(
        _tanh_body,
        grid=(_N // _TC_BR,),
        in_specs=[pl.BlockSpec((_TC_BR, _D), lambda i: (i, 0))],
        out_specs=pl.BlockSpec((_TC_BR, _D), lambda i: (i, 0)),
        out_shape=jax.ShapeDtypeStruct((_N, _D), jnp.float32),
    )(values)

    partials = _sc_scatter(t, widx, wflat)

    out = pl.pallas_call(
        _add_body,
        grid=(_N // _TC_BR,),
        in_specs=[
            pl.BlockSpec((1, _TC_BR, _D), lambda i: (0, i, 0)),
            pl.BlockSpec((1, _TC_BR, _D), lambda i: (1, i, 0)),
        ],
        out_specs=pl.BlockSpec((_TC_BR, _D), lambda i: (i, 0)),
        out_shape=jax.ShapeDtypeStruct((_N, _D), jnp.float32),
    )(partials, partials)
    return out
